# baseline (speedup 1.0000x reference)
import functools

import jax
import jax.numpy as jnp
from jax import lax
from jax.experimental import pallas as pl
from jax.experimental.pallas import tpu as pltpu
from jax.experimental.pallas import tpu_sc as plsc

C = 16
H = 512
W = 512
B = 65536
D_FEAT = 16
HW = H * W

_ROWS = 512
_LANES = 128

_NC = 2
_NS = 16
_NW = _NC * _NS
_BPW = B // _NW


def _proj_body(m_ref, homo_ref, idx_ref, mask_ref):
    uvw = lax.dot_general(m_ref[...], homo_ref[...], (((1,), (0,)), ((), ())))
    for c in range(C):
        u = uvw[3 * c]
        v = uvw[3 * c + 1]
        w = uvw[3 * c + 2]
        x = u / w
        y = v / w
        ix = x.astype(jnp.int32)
        iy = y.astype(jnp.int32)
        mk = (ix >= 0) & (iy >= 0) & (iy < W) & (ix < H)
        idx_ref[c] = (jnp.where(mk, ix * W + iy, 0) + c * HW).reshape(_ROWS, _LANES)
        mask_ref[c] = mk.astype(jnp.int32).reshape(_ROWS, _LANES)


def _project(positions, M):
    homo = jnp.concatenate(
        [positions.T, jnp.ones((1, B), positions.dtype)], axis=0)
    m48 = M.reshape(C * 3, 4)
    idx, mask = pl.pallas_call(
        _proj_body,
        out_shape=(
            jax.ShapeDtypeStruct((C, _ROWS, _LANES), jnp.int32),
            jax.ShapeDtypeStruct((C, _ROWS, _LANES), jnp.int32),
        ),
        in_specs=[
            pl.BlockSpec(memory_space=pltpu.VMEM),
            pl.BlockSpec(memory_space=pltpu.VMEM),
        ],
    )(m48, homo)
    return idx.reshape(C * B), mask.reshape(C, B)


def _gather_body(idx_hbm, img_hbm, feat_hbm, out_img, out_feat,
                 idx_v, img_v, feat_v, sem_a, sem_b):
    wid = lax.axis_index("s") * _NC + lax.axis_index("c")
    b0 = wid * _BPW
    for c in range(C):
        pltpu.sync_copy(idx_hbm.at[pl.ds(c * B + b0, _BPW)], idx_v)
        cp_img = pltpu.async_copy(img_hbm.at[idx_v], img_v, sem_a)
        cp_feat = pltpu.async_copy(feat_hbm.at[idx_v], feat_v, sem_b)
        cp_img.wait()
        cp_feat.wait()
        pltpu.sync_copy(img_v.at[:, pl.ds(0, 3)], out_img.at[pl.ds(b0, _BPW), c])
        pltpu.sync_copy(feat_v, out_feat.at[pl.ds(b0, _BPW), c])


@functools.cache
def _gather():
    return pl.kernel(
        _gather_body,
        out_type=(
            jax.ShapeDtypeStruct((B, C, 3), jnp.float32),
            jax.ShapeDtypeStruct((B, C, D_FEAT), jnp.float32),
        ),
        mesh=plsc.VectorSubcoreMesh(core_axis_name="c", subcore_axis_name="s"),
        compiler_params=pltpu.CompilerParams(use_tc_tiling_on_sc=False),
        scratch_types=[
            pltpu.VMEM((_BPW,), jnp.int32),
            pltpu.VMEM((_BPW, 8), jnp.float32),
            pltpu.VMEM((_BPW, D_FEAT), jnp.float32),
            pltpu.SemaphoreType.DMA,
            pltpu.SemaphoreType.DMA,
        ],
    )


def kernel(positions, images, features, M):
    idx, mask = _project(positions, M)
    img_flat = jnp.pad(images.reshape(C * HW, 3), ((0, 0), (0, 5)))
    feat_flat = features.reshape(C * HW, D_FEAT)
    out_img, out_feat = _gather()(idx, img_flat, feat_flat)
    masks_t = mask.T != 0
    return out_img, out_feat, masks_t


# speedup vs baseline: 1.6203x; 1.6203x over previous
import functools

import jax
import jax.numpy as jnp
from jax import lax
from jax.experimental import pallas as pl
from jax.experimental.pallas import tpu as pltpu
from jax.experimental.pallas import tpu_sc as plsc

C = 16
H = 512
W = 512
B = 65536
D_FEAT = 16
HW = H * W
CHW = C * HW

_ROWS = 512
_LANES = 128

_NC = 2
_NS = 16
_NW = _NC * _NS
_BPW = B // _NW

_HB = 8


def _proj_body(m_ref, homo_ref, idx_ref, mask_ref):
    uvw = lax.dot_general(m_ref[...], homo_ref[...], (((1,), (0,)), ((), ())))
    for c in range(C):
        u = uvw[3 * c]
        v = uvw[3 * c + 1]
        w = uvw[3 * c + 2]
        x = u / w
        y = v / w
        ix = x.astype(jnp.int32)
        iy = y.astype(jnp.int32)
        mk = (ix >= 0) & (iy >= 0) & (iy < W) & (ix < H)
        idx_ref[c] = (jnp.where(mk, ix * W + iy, 0) + c * HW).reshape(_ROWS, _LANES)
        mask_ref[c] = mk.astype(jnp.int32).reshape(_ROWS, _LANES)


def _project(positions, M):
    homo = jnp.concatenate(
        [positions.T, jnp.ones((1, B), positions.dtype)], axis=0)
    m48 = M.reshape(C * 3, 4)
    idx, mask = pl.pallas_call(
        _proj_body,
        out_shape=(
            jax.ShapeDtypeStruct((C, _ROWS, _LANES), jnp.int32),
            jax.ShapeDtypeStruct((C, _ROWS, _LANES), jnp.int32),
        ),
        in_specs=[
            pl.BlockSpec(memory_space=pltpu.VMEM),
            pl.BlockSpec(memory_space=pltpu.VMEM),
        ],
    )(m48, homo)
    return idx.reshape(C * B), mask.reshape(C, B)


def _repack_body(feat_ref, img_ref, tabf_ref, tabi_ref):
    f = feat_ref[0]
    m = img_ref[0]
    tabf_ref[...] = jnp.transpose(f, (0, 2, 1)).reshape(_HB * W, D_FEAT)
    tabi_ref[:, 0:3] = jnp.transpose(m.reshape(3, _HB * W), (1, 0))


def _repack(features, images):
    featv = jnp.transpose(features, (0, 1, 3, 2))
    imgv = jnp.transpose(images, (0, 3, 1, 2))
    return pl.pallas_call(
        _repack_body,
        grid=(C, H // _HB),
        out_shape=(
            jax.ShapeDtypeStruct((CHW, D_FEAT), jnp.float32),
            jax.ShapeDtypeStruct((CHW, 8), jnp.float32),
        ),
        in_specs=[
            pl.BlockSpec((1, _HB, D_FEAT, W), lambda c, h: (c, h, 0, 0)),
            pl.BlockSpec((1, 3, _HB, W), lambda c, h: (c, 0, h, 0)),
        ],
        out_specs=(
            pl.BlockSpec((_HB * W, D_FEAT), lambda c, h: (c * (H // _HB) + h, 0)),
            pl.BlockSpec((_HB * W, 8), lambda c, h: (c * (H // _HB) + h, 0)),
        ),
    )(featv, imgv)


def _gather_body(idx_hbm, tabf_hbm, tabi_hbm, outf_cm, outimg_p,
                 idx_v, buff, bufi, bufp, sem_f, sem_i):
    wid = lax.axis_index("s") * _NC + lax.axis_index("c")
    b0 = wid * _BPW
    lanes = lax.iota(jnp.int32, 16)
    for c in range(C):
        pltpu.sync_copy(idx_hbm.at[pl.ds(c * B + b0, _BPW)], idx_v)
        cp_f = pltpu.async_copy(tabf_hbm.at[idx_v], buff, sem_f)
        cp_i = pltpu.async_copy(tabi_hbm.at[idx_v], bufi, sem_i)
        cp_f.wait()
        pltpu.sync_copy(buff, outf_cm.at[c, pl.ds(b0, _BPW)])
        cp_i.wait()

        def extract(j, _):
            rows = j * 16 + lanes
            for ch in range(3):
                vals = plsc.load_gather(bufi, [rows, jnp.full((16,), ch, jnp.int32)])
                bufp[ch, pl.ds(j * 16, 16)] = vals
            return 0

        lax.fori_loop(0, _BPW // 16, extract, 0)
        for ch in range(3):
            pltpu.sync_copy(bufp.at[ch], outimg_p.at[ch, c, pl.ds(b0, _BPW)])


@functools.cache
def _gather():
    return pl.kernel(
        _gather_body,
        out_type=(
            jax.ShapeDtypeStruct((C, B, D_FEAT), jnp.float32),
            jax.ShapeDtypeStruct((3, C, B), jnp.float32),
        ),
        mesh=plsc.VectorSubcoreMesh(core_axis_name="c", subcore_axis_name="s"),
        compiler_params=pltpu.CompilerParams(
            use_tc_tiling_on_sc=False, needs_layout_passes=False),
        scratch_types=[
            pltpu.VMEM((_BPW,), jnp.int32),
            pltpu.VMEM((_BPW, D_FEAT), jnp.float32),
            pltpu.VMEM((_BPW, 8), jnp.float32),
            pltpu.VMEM((3, _BPW), jnp.float32),
            pltpu.SemaphoreType.DMA,
            pltpu.SemaphoreType.DMA,
        ],
    )


def _tr_body(in_ref, out_ref):
    out_ref[0] = in_ref[0].T


def _feat_transpose(outf_cm):
    nb = 4096
    return pl.pallas_call(
        _tr_body,
        grid=(C, B // nb),
        out_shape=jax.ShapeDtypeStruct((C, D_FEAT, B), jnp.float32),
        in_specs=[pl.BlockSpec((1, nb, D_FEAT), lambda c, b: (c, b, 0))],
        out_specs=pl.BlockSpec((1, D_FEAT, nb), lambda c, b: (c, 0, b)),
    )(outf_cm)


def kernel(positions, images, features, M):
    idx, mask = _project(positions, M)
    tabf, tabi = _repack(features, images)
    outf_cm, outimg_p = _gather()(idx, tabf, tabi)
    outf_t = _feat_transpose(outf_cm)
    out_feat = jnp.transpose(outf_t, (2, 0, 1))
    out_img = jnp.transpose(outimg_p, (2, 1, 0))
    masks_t = mask.T != 0
    return out_img, out_feat, masks_t
